# baseline (speedup 1.0000x reference)
import functools

import jax
import jax.numpy as jnp
from jax import lax
from jax.experimental import pallas as pl
from jax.experimental.pallas import tpu as pltpu
from jax.experimental.pallas import tpu_sc as plsc

BATCH = 16384
EMBED_DIM = 64

_info = plsc.get_sparse_core_info()
_NC, _NS = _info.num_cores, _info.num_subcores
_NW = _NC * _NS
_BPW = BATCH // _NW

_mesh = plsc.VectorSubcoreMesh(core_axis_name="c", subcore_axis_name="s")


@functools.partial(
    pl.kernel,
    mesh=_mesh,
    compiler_params=pltpu.CompilerParams(use_tc_tiling_on_sc=False),
    out_type=[
        jax.ShapeDtypeStruct((BATCH, EMBED_DIM), jnp.float32),
        jax.ShapeDtypeStruct((BATCH, EMBED_DIM), jnp.float32),
    ],
    scratch_types=[
        pltpu.VMEM((_BPW,), jnp.int32),
        pltpu.VMEM((_BPW,), jnp.int32),
        pltpu.VMEM((_BPW, EMBED_DIM), jnp.float32),
        pltpu.VMEM((_BPW, EMBED_DIM), jnp.float32),
        pltpu.SemaphoreType.DMA,
        pltpu.SemaphoreType.DMA,
    ],
)
def _dual_gather(sid_hbm, tid_hbm, w_acoustic_hbm, w_vocoder_hbm,
                 out_acoustic_hbm, out_vocoder_hbm,
                 idx_a, idx_v, rows_a, rows_v, sem_a, sem_v):
    wid = lax.axis_index("s") * _NC + lax.axis_index("c")
    base = wid * _BPW
    pltpu.sync_copy(sid_hbm.at[pl.ds(base, _BPW)], idx_a)
    pltpu.sync_copy(tid_hbm.at[pl.ds(base, _BPW)], idx_v)
    cp_a = pltpu.async_copy(w_acoustic_hbm.at[idx_a], rows_a, sem_a)
    cp_v = pltpu.async_copy(w_vocoder_hbm.at[idx_v], rows_v, sem_v)
    cp_a.wait()
    pltpu.sync_copy(rows_a, out_acoustic_hbm.at[pl.ds(base, _BPW)])
    cp_v.wait()
    pltpu.sync_copy(rows_v, out_vocoder_hbm.at[pl.ds(base, _BPW)])


def kernel(speaker_id, target_speaker_id, speaker_embed_weight,
           vocoder_embed_weight):
    return tuple(_dual_gather(speaker_id, target_speaker_id,
                              speaker_embed_weight, vocoder_embed_weight))


# speedup vs baseline: 2.4673x; 2.4673x over previous
import functools

import jax
import jax.numpy as jnp
from jax import lax
from jax.experimental import pallas as pl
from jax.experimental.pallas import tpu as pltpu
from jax.experimental.pallas import tpu_sc as plsc

BATCH = 16384
EMBED_DIM = 64
VOCAB = 100000

_info = plsc.get_sparse_core_info()
_NC, _NS, _NL = _info.num_cores, _info.num_subcores, _info.num_lanes
_NW = _NC * _NS
_ROWS_PER_W = EMBED_DIM // _NW
_CHUNK = 8192
_UNROLL = 8

_mesh = plsc.VectorSubcoreMesh(core_axis_name="c", subcore_axis_name="s")


@functools.partial(
    pl.kernel,
    mesh=_mesh,
    compiler_params=pltpu.CompilerParams(needs_layout_passes=False),
    out_type=[
        jax.ShapeDtypeStruct((EMBED_DIM, BATCH), jnp.float32),
        jax.ShapeDtypeStruct((EMBED_DIM, BATCH), jnp.float32),
    ],
    scratch_types=[
        pltpu.VMEM((VOCAB,), jnp.float32),
        pltpu.VMEM((BATCH,), jnp.int32),
        pltpu.VMEM((_CHUNK,), jnp.float32),
        pltpu.SemaphoreType.DMA,
    ],
)
def _lane_gather(sid_hbm, tid_hbm, w1t_hbm, w2t_hbm, o1t_hbm, o2t_hbm,
                 row_v, idx_v, out_v, sem):
    wid = lax.axis_index("c") * _NS + lax.axis_index("s")

    def gather_chunk(base):
        def body(i, _):
            off = i * (_NL * _UNROLL)
            for k in range(_UNROLL):
                ids = idx_v[pl.ds(base + off + k * _NL, _NL)]
                out_v[pl.ds(off + k * _NL, _NL)] = plsc.load_gather(
                    row_v, [ids])
            return 0
        lax.fori_loop(0, _CHUNK // (_NL * _UNROLL), body, 0)

    def do_table(idx_hbm, wt_hbm, ot_hbm):
        pltpu.sync_copy(idx_hbm, idx_v)
        for r in range(_ROWS_PER_W):
            d = wid * _ROWS_PER_W + r
            pltpu.sync_copy(wt_hbm.at[d], row_v)
            for c in range(BATCH // _CHUNK):
                gather_chunk(c * _CHUNK)
                pltpu.sync_copy(out_v, ot_hbm.at[d, pl.ds(c * _CHUNK, _CHUNK)])

    do_table(sid_hbm, w1t_hbm, o1t_hbm)
    do_table(tid_hbm, w2t_hbm, o2t_hbm)


def kernel(speaker_id, target_speaker_id, speaker_embed_weight,
           vocoder_embed_weight):
    o1t, o2t = _lane_gather(speaker_id, target_speaker_id,
                            speaker_embed_weight.T, vocoder_embed_weight.T)
    return (o1t.T, o2t.T)
